# baseline (speedup 1.0000x reference)
import functools

import jax
import jax.numpy as jnp
from jax import lax
from jax.experimental import pallas as pl
from jax.experimental.pallas import tpu as pltpu
from jax.experimental.pallas import tpu_sc as plsc

B, L, D = 4096, 200, 64
NC, NS = 2, 16
NW = NC * NS
ROWS = B * L
RPW = ROWS // NW
CHUNK = 128
NCHUNK = RPW // CHUNK


def _body(x0_hbm, x1_hbm, out_hbm, idx_v, rows_v, sem):
    c = lax.axis_index("c")
    s = lax.axis_index("s")
    w = c * NS + s
    base = w * RPW

    pltpu.sync_copy(x1_hbm.at[w], idx_v)

    lane = lax.iota(jnp.int32, 16)
    l_vec = jnp.full((16,), L, dtype=jnp.int32)

    def off_body(i, carry):
        row_start = base + i * CHUNK
        for jj in range(CHUNK // 16):
            pos = lane + jnp.full((16,), row_start + jj * 16, dtype=jnp.int32)
            off = lax.mul(lax.div(pos, l_vec), l_vec)
            idx_v[i, pl.ds(jj * 16, 16)] = idx_v[i, pl.ds(jj * 16, 16)] + off
        return carry

    lax.fori_loop(0, NCHUNK, off_body, 0)

    def g_body(j, carry):
        pltpu.async_copy(x0_hbm.at[idx_v.at[j]], rows_v, sem).wait()
        pltpu.sync_copy(rows_v, out_hbm.at[w, j])
        return carry

    lax.fori_loop(0, NCHUNK, g_body, 0)


@jax.jit
def kernel(x0, x1):
    x0f = x0.reshape(ROWS, D)
    x1f = x1.astype(jnp.int32).reshape(NW, NCHUNK, CHUNK)
    mesh = plsc.VectorSubcoreMesh(core_axis_name="c", subcore_axis_name="s")
    out = pl.kernel(
        _body,
        mesh=mesh,
        out_type=jax.ShapeDtypeStruct((NW, NCHUNK, CHUNK, D), jnp.float32),
        scratch_types=[
            pltpu.VMEM((NCHUNK, CHUNK), jnp.int32),
            pltpu.VMEM((CHUNK, D), jnp.float32),
            pltpu.SemaphoreType.DMA,
        ],
        compiler_params=pltpu.CompilerParams(use_tc_tiling_on_sc=False),
    )(x0f, x1f)
    return out.reshape(B, L, D)
